# baseline (device time: 28933 ns/iter reference)
import functools

import jax
import jax.numpy as jnp
from jax import lax
from jax.experimental import pallas as pl
from jax.experimental.pallas import tpu as pltpu

N_DEV = 4


def kernel(x, w_mat):
    m_per, k_dim = x.shape
    n = w_mat.shape[1]
    n_per = n // N_DEV

    def body(x_ref, w_ref, out_ref, send_buf, recv_buf, send_sems, recv_sems):
        my = lax.axis_index("i")

        barrier_sem = pltpu.get_barrier_semaphore()
        for hop in range(1, N_DEV):
            pl.semaphore_signal(
                barrier_sem, inc=1,
                device_id=((my + hop) % N_DEV,),
                device_id_type=pl.DeviceIdType.MESH,
            )
        pl.semaphore_wait(barrier_sem, N_DEV - 1)

        x_val = x_ref[:, :]
        sends = []
        for hop in range(1, N_DEV):
            dst = (my + hop) % N_DEV
            w_blk = w_ref[:, pl.ds(dst * n_per, n_per)]
            y = jnp.dot(x_val, w_blk, preferred_element_type=jnp.float32)
            send_buf[hop - 1] = y.astype(jnp.bfloat16)
            rdma = pltpu.make_async_remote_copy(
                src_ref=send_buf.at[hop - 1],
                dst_ref=recv_buf.at[hop - 1],
                send_sem=send_sems.at[hop - 1],
                recv_sem=recv_sems.at[hop - 1],
                device_id=(dst,),
                device_id_type=pl.DeviceIdType.MESH,
            )
            rdma.start()
            sends.append(rdma)

        w_blk = w_ref[:, pl.ds(my * n_per, n_per)]
        y = jnp.dot(x_val, w_blk, preferred_element_type=jnp.float32)
        out_ref[pl.ds(my * m_per, m_per), :] = y * jax.nn.sigmoid(y)

        for hop in range(1, N_DEV):
            sends[hop - 1].wait_recv()
            src_dev = (my - hop) % N_DEV
            y = recv_buf[hop - 1].astype(jnp.float32)
            out_ref[pl.ds(src_dev * m_per, m_per), :] = y * jax.nn.sigmoid(y)

        for hop in range(1, N_DEV):
            sends[hop - 1].wait_send()

    return pl.pallas_call(
        body,
        out_shape=jax.ShapeDtypeStruct((N_DEV * m_per, n_per), jnp.float32),
        in_specs=[
            pl.BlockSpec(memory_space=pltpu.VMEM),
            pl.BlockSpec(memory_space=pltpu.VMEM),
        ],
        out_specs=pl.BlockSpec(memory_space=pltpu.VMEM),
        scratch_shapes=[
            pltpu.VMEM((N_DEV - 1, m_per, n_per), jnp.bfloat16),
            pltpu.VMEM((N_DEV - 1, m_per, n_per), jnp.bfloat16),
            pltpu.SemaphoreType.DMA((N_DEV - 1,)),
            pltpu.SemaphoreType.DMA((N_DEV - 1,)),
        ],
        compiler_params=pltpu.CompilerParams(collective_id=0),
    )(x, w_mat)
